# Initial kernel scaffold; baseline (speedup 1.0000x reference)
import jax
import jax.numpy as jnp
from jax.experimental import pallas as pl


def kernel(x, edge_index, W, b):
    raise NotImplementedError("write your pallas kernel here")


# speedup vs baseline: 9.0373x; 9.0373x over previous
import functools

import jax
import jax.numpy as jnp
from jax import lax
from jax.experimental import pallas as pl
from jax.experimental.pallas import tpu as pltpu
from jax.experimental.pallas import tpu_sc as plsc

ALPHA = 0.1
KSTEPS = 5
NSUB = 16
NCORE = 2
LANES = 16
CHUNK = 128

_SC_PARAMS = pltpu.CompilerParams(
    needs_layout_passes=False, use_tc_tiling_on_sc=False)


def _sc_degree(n_pad, rpt, chunks):
    mesh = plsc.VectorSubcoreMesh(core_axis_name="c", subcore_axis_name="s")

    @functools.partial(
        pl.kernel,
        out_type=jax.ShapeDtypeStruct((n_pad,), jnp.float32),
        mesh=mesh,
        compiler_params=_SC_PARAMS,
        scratch_types=[
            pltpu.VMEM_SHARED((n_pad,), jnp.float32),
            pltpu.VMEM((CHUNK,), jnp.int32),
            pltpu.VMEM((CHUNK,), jnp.float32),
        ],
    )
    def deg_kernel(dst_hbm, deg_out, DEG, didx, ones_t):
        c = lax.axis_index("c")
        s = lax.axis_index("s")
        row0 = s * rpt
        ones16 = jnp.full((LANES,), 1.0, jnp.float32)

        @pl.when(c == 0)
        def _():
            def f_ones(i, carry):
                ones_t[pl.ds(i * LANES, LANES)] = ones16
                return carry
            lax.fori_loop(0, CHUNK // LANES, f_ones, 0)

            def f_deginit(j, carry):
                pltpu.sync_copy(ones_t, DEG.at[pl.ds(row0 + j * CHUNK, CHUNK)])
                return carry
            lax.fori_loop(0, rpt // CHUNK, f_deginit, 0)
            plsc.subcore_barrier()

            e0 = s * (chunks * CHUNK)

            def f_deg(ch, carry):
                off = e0 + ch * CHUNK
                pltpu.sync_copy(dst_hbm.at[pl.ds(off, CHUNK)], didx)
                pltpu.sync_copy(ones_t, DEG.at[didx], add=True)
                return carry
            lax.fori_loop(0, chunks, f_deg, 0)
            plsc.subcore_barrier()
            pltpu.sync_copy(DEG.at[pl.ds(row0, rpt)],
                            deg_out.at[pl.ds(row0, rpt)])

    return deg_kernel


def _sc_propagate(n_pad, f_half, rpt, chunks):
    mesh = plsc.VectorSubcoreMesh(core_axis_name="c", subcore_axis_name="s")

    @functools.partial(
        pl.kernel,
        out_type=jax.ShapeDtypeStruct((KSTEPS, NCORE, n_pad, f_half),
                                      jnp.float32),
        mesh=mesh,
        compiler_params=_SC_PARAMS,
        scratch_types=[
            pltpu.VMEM_SHARED((n_pad, f_half), jnp.float32),
            pltpu.VMEM_SHARED((n_pad, f_half), jnp.float32),
            pltpu.VMEM((CHUNK, f_half), jnp.float32),
            pltpu.VMEM((CHUNK, f_half), jnp.float32),
            pltpu.VMEM((CHUNK,), jnp.int32),
            pltpu.VMEM((CHUNK,), jnp.int32),
            pltpu.VMEM((rpt,), jnp.float32),
            pltpu.VMEM((rpt,), jnp.float32),
        ],
    )
    def prop(x2_hbm, src_hbm, dst_hbm, dinv_hbm, recip_hbm, u_out,
             U, Y, ytile, rows, sidx, didx, dinv_v, recip_v):
        c = lax.axis_index("c")
        s = lax.axis_index("s")
        row0 = s * rpt
        e0 = s * (chunks * CHUNK)
        nsub = rpt // CHUNK

        pltpu.sync_copy(dinv_hbm.at[pl.ds(row0, rpt)], dinv_v)
        pltpu.sync_copy(recip_hbm.at[pl.ds(row0, rpt)], recip_v)

        def f_x(j, carry):
            r0 = row0 + j * CHUNK
            pltpu.sync_copy(x2_hbm.at[c, pl.ds(r0, CHUNK)], ytile)

            def f_row(r, carry2):
                idx = jnp.full((LANES,), j * CHUNK + r, jnp.int32)
                dv = plsc.load_gather(dinv_v, [idx])
                for c2 in range(f_half // LANES):
                    sl = pl.ds(c2 * LANES, LANES)
                    ytile[r, sl] = ytile[r, sl] * dv
                return carry2
            lax.fori_loop(0, CHUNK, f_row, 0)
            pltpu.sync_copy(ytile, U.at[pl.ds(r0, CHUNK)])
            pltpu.sync_copy(ytile, Y.at[pl.ds(r0, CHUNK)])
            return carry
        lax.fori_loop(0, nsub, f_x, 0)
        plsc.subcore_barrier()

        for k in range(1, KSTEPS + 1):
            def f_edge(ch, carry):
                off = e0 + ch * CHUNK
                pltpu.sync_copy(src_hbm.at[pl.ds(off, CHUNK)], sidx)
                pltpu.sync_copy(dst_hbm.at[pl.ds(off, CHUNK)], didx)
                pltpu.sync_copy(U.at[sidx], rows)
                pltpu.sync_copy(rows, Y.at[didx], add=True)
                return carry
            lax.fori_loop(0, chunks, f_edge, 0)
            plsc.subcore_barrier()

            def f_ew(j, carry):
                r0 = row0 + j * CHUNK
                pltpu.sync_copy(Y.at[pl.ds(r0, CHUNK)], ytile)

                def f_row(r, carry2):
                    idx = jnp.full((LANES,), j * CHUNK + r, jnp.int32)
                    rv = plsc.load_gather(recip_v, [idx])
                    for c2 in range(f_half // LANES):
                        sl = pl.ds(c2 * LANES, LANES)
                        ytile[r, sl] = ytile[r, sl] * rv
                    return carry2
                lax.fori_loop(0, CHUNK, f_row, 0)
                pltpu.sync_copy(ytile, u_out.at[k - 1, c, pl.ds(r0, CHUNK)])
                if k < KSTEPS:
                    pltpu.sync_copy(ytile, U.at[pl.ds(r0, CHUNK)])
                    pltpu.sync_copy(ytile, Y.at[pl.ds(r0, CHUNK)])
                return carry
            lax.fori_loop(0, nsub, f_ew, 0)
            if k < KSTEPS:
                plsc.subcore_barrier()

    return prop


def _tc_linear(n_pad, d_in, d_out, blk):
    f_half = d_in // 2
    coef = (1.0 - ALPHA) / KSTEPS

    def body(x_ref, u_ref, deg_ref, w_ref, b_ref, o_ref):
        dg = jnp.sqrt(deg_ref[...])
        us = u_ref[...]
        ssum = us[0]
        for k in range(1, KSTEPS):
            ssum = ssum + us[k]
        s_full = jnp.concatenate([ssum[0], ssum[1]], axis=1)
        xo = ALPHA * x_ref[...] + coef * dg * s_full
        o_ref[...] = (
            jnp.dot(xo, w_ref[...], preferred_element_type=jnp.float32)
            + b_ref[...]
        )

    return pl.pallas_call(
        body,
        grid=(n_pad // blk,),
        in_specs=[
            pl.BlockSpec((blk, d_in), lambda i: (i, 0)),
            pl.BlockSpec((KSTEPS, NCORE, blk, f_half), lambda i: (0, 0, i, 0)),
            pl.BlockSpec((blk, 1), lambda i: (i, 0)),
            pl.BlockSpec((d_in, d_out), lambda i: (0, 0)),
            pl.BlockSpec((1, d_out), lambda i: (0, 0)),
        ],
        out_specs=pl.BlockSpec((blk, d_out), lambda i: (i, 0)),
        out_shape=jax.ShapeDtypeStruct((n_pad, d_out), jnp.float32),
    )


def kernel(x, edge_index, W, b):
    n, d_in = x.shape
    d_out = W.shape[1]
    e = edge_index.shape[1]
    f_half = d_in // NCORE

    rpt = -(-n // (NSUB * CHUNK)) * CHUNK
    n_pad = NSUB * rpt
    chunks = -(-e // (NSUB * CHUNK))
    e_pad = NSUB * chunks * CHUNK

    src = edge_index[0]
    dst = edge_index[1]
    npad_e = e_pad - e
    if npad_e:
        fill = jnp.arange(npad_e, dtype=jnp.int32)
        src = jnp.concatenate([src, fill % n])
        dst = jnp.concatenate([dst, n + fill % (n_pad - n)])

    deg = _sc_degree(n_pad, rpt, chunks)(dst)
    dinv = lax.rsqrt(deg)
    recip = dinv * dinv

    x2 = jnp.stack([x[:, :f_half], x[:, f_half:]])
    x2 = jnp.pad(x2, ((0, 0), (0, n_pad - n), (0, 0)))

    u_all = _sc_propagate(n_pad, f_half, rpt, chunks)(x2, src, dst, dinv, recip)

    x_pad = jnp.pad(x, ((0, n_pad - n), (0, 0)))
    out = _tc_linear(n_pad, d_in, d_out, 1024)(
        x_pad, u_all, deg[:, None], W, b[None, :]
    )
    return out[:n]


# speedup vs baseline: 12.6107x; 1.3954x over previous
import functools

import jax
import jax.numpy as jnp
from jax import lax
from jax.experimental import pallas as pl
from jax.experimental.pallas import tpu as pltpu
from jax.experimental.pallas import tpu_sc as plsc

ALPHA = 0.1
KSTEPS = 5
NSUB = 16
NCORE = 2
LANES = 16
CHUNK = 128

_SC_PARAMS = pltpu.CompilerParams(
    needs_layout_passes=False, use_tc_tiling_on_sc=False)


def _sc_degree(n_pad, rpt, chunks):
    mesh = plsc.VectorSubcoreMesh(core_axis_name="c", subcore_axis_name="s")

    @functools.partial(
        pl.kernel,
        out_type=jax.ShapeDtypeStruct((n_pad,), jnp.float32),
        mesh=mesh,
        compiler_params=_SC_PARAMS,
        scratch_types=[
            pltpu.VMEM_SHARED((n_pad,), jnp.float32),
            pltpu.VMEM((2, CHUNK), jnp.int32),
            pltpu.VMEM((CHUNK,), jnp.float32),
        ],
    )
    def deg_kernel(ei_hbm, deg_out, DEG, didx, ones_t):
        c = lax.axis_index("c")
        s = lax.axis_index("s")
        row0 = s * rpt
        ones16 = jnp.full((LANES,), 1.0, jnp.float32)

        @pl.when(c == 0)
        def _():
            def f_ones(i, carry):
                ones_t[pl.ds(i * LANES, LANES)] = ones16
                return carry
            lax.fori_loop(0, CHUNK // LANES, f_ones, 0)

            def f_deginit(j, carry):
                pltpu.sync_copy(ones_t, DEG.at[pl.ds(row0 + j * CHUNK, CHUNK)])
                return carry
            lax.fori_loop(0, rpt // CHUNK, f_deginit, 0)
            plsc.subcore_barrier()

            ch0 = s * chunks

            def f_deg(ch, carry):
                pltpu.sync_copy(ei_hbm.at[ch0 + ch], didx)
                pltpu.sync_copy(ones_t, DEG.at[didx.at[1]], add=True)
                return carry
            lax.fori_loop(0, chunks, f_deg, 0)
            plsc.subcore_barrier()
            pltpu.sync_copy(DEG.at[pl.ds(row0, rpt)],
                            deg_out.at[pl.ds(row0, rpt)])

    return deg_kernel


def _sc_propagate(n_pad, f_half, rpt, chunks):
    mesh = plsc.VectorSubcoreMesh(core_axis_name="c", subcore_axis_name="s")

    @functools.partial(
        pl.kernel,
        out_type=jax.ShapeDtypeStruct((KSTEPS, NCORE, n_pad, f_half),
                                      jnp.float32),
        mesh=mesh,
        compiler_params=_SC_PARAMS,
        scratch_types=[
            pltpu.VMEM_SHARED((n_pad, f_half), jnp.float32),
            pltpu.VMEM_SHARED((n_pad, f_half), jnp.float32),
            pltpu.VMEM((CHUNK, f_half), jnp.float32),
            pltpu.VMEM((CHUNK, f_half), jnp.float32),
            pltpu.VMEM((CHUNK, f_half), jnp.float32),
            pltpu.VMEM((2, CHUNK), jnp.int32),
            pltpu.VMEM((2, CHUNK), jnp.int32),
            pltpu.VMEM((rpt,), jnp.float32),
            pltpu.VMEM((rpt,), jnp.float32),
            pltpu.SemaphoreType.DMA,
            pltpu.SemaphoreType.DMA,
        ],
    )
    def prop(x2_hbm, ei_hbm, dinv_hbm, recip_hbm, u_out,
             U, Y, ytile, rows_a, rows_b, idx_a, idx_b,
             dinv_v, recip_v, gsem_a, gsem_b):
        c = lax.axis_index("c")
        s = lax.axis_index("s")
        row0 = s * rpt
        ch0 = s * chunks
        nsub = rpt // CHUNK

        pltpu.sync_copy(dinv_hbm.at[pl.ds(row0, rpt)], dinv_v)
        pltpu.sync_copy(recip_hbm.at[pl.ds(row0, rpt)], recip_v)

        def f_x(j, carry):
            r0 = row0 + j * CHUNK
            pltpu.sync_copy(x2_hbm.at[c, pl.ds(r0, CHUNK)], ytile)

            def f_row(r, carry2):
                idx = jnp.full((LANES,), j * CHUNK + r, jnp.int32)
                dv = plsc.load_gather(dinv_v, [idx])
                for c2 in range(f_half // LANES):
                    sl = pl.ds(c2 * LANES, LANES)
                    ytile[r, sl] = ytile[r, sl] * dv
                return carry2
            lax.fori_loop(0, CHUNK, f_row, 0)
            pltpu.sync_copy(ytile, U.at[pl.ds(r0, CHUNK)])
            pltpu.sync_copy(ytile, Y.at[pl.ds(r0, CHUNK)])
            return carry
        lax.fori_loop(0, nsub, f_x, 0)
        plsc.subcore_barrier()


        ga = pltpu.make_async_copy(U.at[idx_a.at[0]], rows_a, gsem_a)
        gb = pltpu.make_async_copy(U.at[idx_b.at[0]], rows_b, gsem_b)

        for k in range(1, KSTEPS + 1):
            pltpu.sync_copy(ei_hbm.at[ch0], idx_a)
            ga.start()

            def f_pair(i2, carry):
                c0 = ch0 + 2 * i2
                pltpu.sync_copy(ei_hbm.at[c0 + 1], idx_b)
                gb.start()
                ga.wait()
                pltpu.sync_copy(rows_a, Y.at[idx_a.at[1]], add=True)
                c2 = jnp.minimum(c0 + 2, ch0 + chunks - 1)
                pltpu.sync_copy(ei_hbm.at[c2], idx_a)
                ga.start()
                gb.wait()
                pltpu.sync_copy(rows_b, Y.at[idx_b.at[1]], add=True)
                return carry
            lax.fori_loop(0, chunks // 2, f_pair, 0)
            ga.wait()
            plsc.subcore_barrier()

            def f_ew(j, carry):
                r0 = row0 + j * CHUNK
                pltpu.sync_copy(Y.at[pl.ds(r0, CHUNK)], ytile)

                def f_row(r, carry2):
                    idx = jnp.full((LANES,), j * CHUNK + r, jnp.int32)
                    rv = plsc.load_gather(recip_v, [idx])
                    for c2 in range(f_half // LANES):
                        sl = pl.ds(c2 * LANES, LANES)
                        ytile[r, sl] = ytile[r, sl] * rv
                    return carry2
                lax.fori_loop(0, CHUNK, f_row, 0)
                pltpu.sync_copy(ytile, u_out.at[k - 1, c, pl.ds(r0, CHUNK)])
                if k < KSTEPS:
                    pltpu.sync_copy(ytile, U.at[pl.ds(r0, CHUNK)])
                    pltpu.sync_copy(ytile, Y.at[pl.ds(r0, CHUNK)])
                return carry
            lax.fori_loop(0, nsub, f_ew, 0)
            if k < KSTEPS:
                plsc.subcore_barrier()

    return prop


def _tc_linear(n_pad, d_in, d_out, blk):
    f_half = d_in // 2
    coef = (1.0 - ALPHA) / KSTEPS

    def body(x_ref, u_ref, deg_ref, w_ref, b_ref, o_ref):
        dg = jnp.sqrt(deg_ref[...])
        us = u_ref[...]
        ssum = us[0]
        for k in range(1, KSTEPS):
            ssum = ssum + us[k]
        s_full = jnp.concatenate([ssum[0], ssum[1]], axis=1)
        xo = ALPHA * x_ref[...] + coef * dg * s_full
        o_ref[...] = (
            jnp.dot(xo, w_ref[...], preferred_element_type=jnp.float32)
            + b_ref[...]
        )

    return pl.pallas_call(
        body,
        grid=(n_pad // blk,),
        in_specs=[
            pl.BlockSpec((blk, d_in), lambda i: (i, 0)),
            pl.BlockSpec((KSTEPS, NCORE, blk, f_half), lambda i: (0, 0, i, 0)),
            pl.BlockSpec((blk, 1), lambda i: (i, 0)),
            pl.BlockSpec((d_in, d_out), lambda i: (0, 0)),
            pl.BlockSpec((1, d_out), lambda i: (0, 0)),
        ],
        out_specs=pl.BlockSpec((blk, d_out), lambda i: (i, 0)),
        out_shape=jax.ShapeDtypeStruct((n_pad, d_out), jnp.float32),
    )


def kernel(x, edge_index, W, b):
    n, d_in = x.shape
    d_out = W.shape[1]
    e = edge_index.shape[1]
    f_half = d_in // NCORE

    rpt = -(-n // (NSUB * CHUNK)) * CHUNK
    n_pad = NSUB * rpt
    chunks = 2 * -(-e // (NSUB * CHUNK * 2))
    e_pad = NSUB * chunks * CHUNK

    src = edge_index[0]
    dst = edge_index[1]
    npad_e = e_pad - e
    if npad_e:
        fill = jnp.arange(npad_e, dtype=jnp.int32)
        src = jnp.concatenate([src, fill % n])
        dst = jnp.concatenate([dst, n + fill % (n_pad - n)])
    ei = jnp.stack([src.reshape(-1, CHUNK), dst.reshape(-1, CHUNK)], axis=1)

    deg = _sc_degree(n_pad, rpt, chunks)(ei)
    dinv = lax.rsqrt(deg)
    recip = dinv * dinv

    x2 = jnp.stack([x[:, :f_half], x[:, f_half:]])
    x2 = jnp.pad(x2, ((0, 0), (0, n_pad - n), (0, 0)))

    u_all = _sc_propagate(n_pad, f_half, rpt, chunks)(x2, ei, dinv, recip)

    x_pad = jnp.pad(x, ((0, n_pad - n), (0, 0)))
    out = _tc_linear(n_pad, d_in, d_out, 1024)(
        x_pad, u_all, deg[:, None], W, b[None, :]
    )
    return out[:n]


# speedup vs baseline: 19.7034x; 1.5624x over previous
import functools

import jax
import jax.numpy as jnp
from jax import lax
from jax.experimental import pallas as pl
from jax.experimental.pallas import tpu as pltpu
from jax.experimental.pallas import tpu_sc as plsc

ALPHA = 0.1
KSTEPS = 5
NSUB = 16
NCORE = 2
LANES = 16
CHUNK = 128

_SC_PARAMS = pltpu.CompilerParams(
    needs_layout_passes=False, use_tc_tiling_on_sc=False)


def _sc_degree(n_pad, rpt, chunks):
    mesh = plsc.VectorSubcoreMesh(core_axis_name="c", subcore_axis_name="s")

    half = -(-chunks // 2)

    @functools.partial(
        pl.kernel,
        out_type=jax.ShapeDtypeStruct((NCORE, n_pad), jnp.float32),
        mesh=mesh,
        compiler_params=_SC_PARAMS,
        scratch_types=[
            pltpu.VMEM_SHARED((n_pad,), jnp.float32),
            pltpu.VMEM((CHUNK,), jnp.int32),
            pltpu.VMEM((CHUNK,), jnp.int32),
            pltpu.VMEM((CHUNK,), jnp.float32),
            pltpu.VMEM((CHUNK,), jnp.float32),
            pltpu.SemaphoreType.DMA,
            pltpu.SemaphoreType.DMA,
        ],
    )
    def deg_kernel(ei_hbm, deg_out, DEG, didx_a, didx_b, ones_t, init_t,
                   isem_a, isem_b):
        c = lax.axis_index("c")
        s = lax.axis_index("s")
        row0 = s * rpt
        ones16 = jnp.full((LANES,), 1.0, jnp.float32)
        init16 = jnp.full((LANES,), jnp.where(c == 0, 1.0, 0.0))

        def f_ones(i, carry):
            ones_t[pl.ds(i * LANES, LANES)] = ones16
            init_t[pl.ds(i * LANES, LANES)] = init16
            return carry
        lax.fori_loop(0, CHUNK // LANES, f_ones, 0)

        def f_deginit(j, carry):
            pltpu.sync_copy(init_t, DEG.at[pl.ds(row0 + j * CHUNK, CHUNK)])
            return carry
        lax.fori_loop(0, rpt // CHUNK, f_deginit, 0)
        plsc.subcore_barrier()

        w0 = s * chunks + c * half
        hc = jnp.where(c == 0, half, chunks - half)
        wlast = s * chunks + chunks - 1
        ia = pltpu.make_async_copy(ei_hbm.at[w0, 1], didx_a, isem_a)
        ia.start()

        def f_deg(i2, carry):
            c0 = w0 + 2 * i2
            ib = pltpu.make_async_copy(
                ei_hbm.at[jnp.minimum(c0 + 1, wlast), 1], didx_b, isem_b)
            ib.start()
            ia.wait()

            @pl.when(c0 < w0 + hc)
            def _():
                pltpu.sync_copy(ones_t, DEG.at[didx_a], add=True)
            ia2 = pltpu.make_async_copy(
                ei_hbm.at[jnp.minimum(c0 + 2, wlast), 1], didx_a, isem_a)
            ia2.start()
            ib.wait()

            @pl.when(c0 + 1 < w0 + hc)
            def _():
                pltpu.sync_copy(ones_t, DEG.at[didx_b], add=True)
            return carry
        lax.fori_loop(0, (half + 1) // 2, f_deg, 0)
        ia.wait()
        plsc.subcore_barrier()
        pltpu.sync_copy(DEG.at[pl.ds(row0, rpt)],
                        deg_out.at[c, pl.ds(row0, rpt)])

    return deg_kernel


def _sc_propagate(n_pad, f_half, rpt, chunks):
    mesh = plsc.VectorSubcoreMesh(core_axis_name="c", subcore_axis_name="s")

    @functools.partial(
        pl.kernel,
        out_type=jax.ShapeDtypeStruct((KSTEPS, NCORE, n_pad, f_half),
                                      jnp.float32),
        mesh=mesh,
        compiler_params=_SC_PARAMS,
        scratch_types=[
            pltpu.VMEM_SHARED((n_pad, f_half), jnp.float32),
            pltpu.VMEM_SHARED((n_pad, f_half), jnp.float32),
            pltpu.VMEM((CHUNK, f_half), jnp.float32),
            pltpu.VMEM((CHUNK, f_half), jnp.float32),
            pltpu.VMEM((CHUNK, f_half), jnp.float32),
            pltpu.VMEM((CHUNK, f_half), jnp.float32),
            pltpu.VMEM((2, CHUNK), jnp.int32),
            pltpu.VMEM((2, CHUNK), jnp.int32),
            pltpu.VMEM((2, CHUNK), jnp.int32),
            pltpu.VMEM((rpt,), jnp.float32),
            pltpu.VMEM((rpt,), jnp.float32),
            pltpu.SemaphoreType.DMA,
            pltpu.SemaphoreType.DMA,
            pltpu.SemaphoreType.DMA,
            pltpu.SemaphoreType.DMA,
            pltpu.SemaphoreType.DMA,
            pltpu.SemaphoreType.DMA,
        ],
    )
    def prop(x2_hbm, ei_hbm, dinv_hbm, recip_hbm, u_out,
             U, Y, ytile, rows_0, rows_1, rows_2, idx_0, idx_1, idx_2,
             dinv_v, recip_v, gsem_0, gsem_1, gsem_2, ssem_0, ssem_1, ssem_2):
        c = lax.axis_index("c")
        s = lax.axis_index("s")
        row0 = s * rpt
        ch0 = s * chunks
        nsub = rpt // CHUNK

        pltpu.sync_copy(dinv_hbm.at[pl.ds(row0, rpt)], dinv_v)
        pltpu.sync_copy(recip_hbm.at[pl.ds(row0, rpt)], recip_v)

        def f_x(j, carry):
            r0 = row0 + j * CHUNK
            pltpu.sync_copy(x2_hbm.at[c, pl.ds(r0, CHUNK)], ytile)

            def f_row(r, carry2):
                idx = jnp.full((LANES,), j * CHUNK + r, jnp.int32)
                dv = plsc.load_gather(dinv_v, [idx])
                for c2 in range(f_half // LANES):
                    sl = pl.ds(c2 * LANES, LANES)
                    ytile[r, sl] = ytile[r, sl] * dv
                return carry2
            lax.fori_loop(0, CHUNK, f_row, 0)
            pltpu.sync_copy(ytile, U.at[pl.ds(r0, CHUNK)])
            pltpu.sync_copy(ytile, Y.at[pl.ds(r0, CHUNK)])
            return carry
        lax.fori_loop(0, nsub, f_x, 0)
        plsc.subcore_barrier()


        rows = (rows_0, rows_1, rows_2)
        idx = (idx_0, idx_1, idx_2)
        gsem = (gsem_0, gsem_1, gsem_2)
        ssem = (ssem_0, ssem_1, ssem_2)
        gd = tuple(pltpu.make_async_copy(U.at[idx[j].at[0]], rows[j], gsem[j])
                   for j in range(3))
        clast = ch0 + chunks - 1

        for k in range(1, KSTEPS + 1):
            pltpu.sync_copy(ei_hbm.at[ch0], idx_0)
            gd[0].start()
            pltpu.sync_copy(ei_hbm.at[ch0 + 1], idx_1)
            gd[1].start()

            def f_tri(i3, carry):
                c0 = ch0 + 3 * i3
                scat = []
                for j in range(3):
                    gd[j].wait()
                    scat.append(pltpu.async_copy(
                        rows[j], Y.at[idx[j].at[1]], ssem[j], add=True))
                    if j > 0:
                        scat[j - 1].wait()
                        jp = j - 1
                    else:
                        jp = 2
                    cn = jnp.minimum(c0 + 2 + j, clast)
                    pltpu.sync_copy(ei_hbm.at[cn], idx[jp])
                    gd[jp].start()
                scat[2].wait()
                return carry
            lax.fori_loop(0, chunks // 3, f_tri, 0)
            gd[0].wait()
            gd[1].wait()
            plsc.subcore_barrier()

            def f_ew(j, carry):
                r0 = row0 + j * CHUNK
                pltpu.sync_copy(Y.at[pl.ds(r0, CHUNK)], ytile)

                def f_row(r, carry2):
                    idx = jnp.full((LANES,), j * CHUNK + r, jnp.int32)
                    rv = plsc.load_gather(recip_v, [idx])
                    for c2 in range(f_half // LANES):
                        sl = pl.ds(c2 * LANES, LANES)
                        ytile[r, sl] = ytile[r, sl] * rv
                    return carry2
                lax.fori_loop(0, CHUNK, f_row, 0)
                pltpu.sync_copy(ytile, u_out.at[k - 1, c, pl.ds(r0, CHUNK)])
                if k < KSTEPS:
                    pltpu.sync_copy(ytile, U.at[pl.ds(r0, CHUNK)])
                    pltpu.sync_copy(ytile, Y.at[pl.ds(r0, CHUNK)])
                return carry
            lax.fori_loop(0, nsub, f_ew, 0)
            if k < KSTEPS:
                plsc.subcore_barrier()

    return prop


def _tc_linear(n_pad, d_in, d_out, blk):
    f_half = d_in // 2
    coef = (1.0 - ALPHA) / KSTEPS

    def body(x_ref, u_ref, deg_ref, w_ref, b_ref, o_ref):
        dg = jnp.sqrt(deg_ref[...])
        us = u_ref[...]
        ssum = us[0]
        for k in range(1, KSTEPS):
            ssum = ssum + us[k]
        s_full = jnp.concatenate([ssum[0], ssum[1]], axis=1)
        xo = ALPHA * x_ref[...] + coef * dg * s_full
        o_ref[...] = (
            jnp.dot(xo, w_ref[...], preferred_element_type=jnp.float32)
            + b_ref[...]
        )

    return pl.pallas_call(
        body,
        grid=(n_pad // blk,),
        in_specs=[
            pl.BlockSpec((blk, d_in), lambda i: (i, 0)),
            pl.BlockSpec((KSTEPS, NCORE, blk, f_half), lambda i: (0, 0, i, 0)),
            pl.BlockSpec((blk, 1), lambda i: (i, 0)),
            pl.BlockSpec((d_in, d_out), lambda i: (0, 0)),
            pl.BlockSpec((1, d_out), lambda i: (0, 0)),
        ],
        out_specs=pl.BlockSpec((blk, d_out), lambda i: (i, 0)),
        out_shape=jax.ShapeDtypeStruct((n_pad, d_out), jnp.float32),
    )


def kernel(x, edge_index, W, b):
    n, d_in = x.shape
    d_out = W.shape[1]
    e = edge_index.shape[1]
    f_half = d_in // NCORE

    rpt = -(-n // (NSUB * CHUNK)) * CHUNK
    n_pad = NSUB * rpt
    chunks = 3 * -(-e // (NSUB * CHUNK * 3))
    e_pad = NSUB * chunks * CHUNK

    src = edge_index[0]
    dst = edge_index[1]
    npad_e = e_pad - e
    if npad_e:
        fill = jnp.arange(npad_e, dtype=jnp.int32)
        src = jnp.concatenate([src, fill % n])
        dst = jnp.concatenate([dst, n + fill % (n_pad - n)])
    ei = jnp.stack([src.reshape(-1, CHUNK), dst.reshape(-1, CHUNK)], axis=1)

    deg2 = _sc_degree(n_pad, rpt, chunks)(ei)
    deg = deg2[0] + deg2[1]
    dinv = lax.rsqrt(deg)
    recip = dinv * dinv

    x2 = jnp.stack([x[:, :f_half], x[:, f_half:]])
    x2 = jnp.pad(x2, ((0, 0), (0, n_pad - n), (0, 0)))

    u_all = _sc_propagate(n_pad, f_half, rpt, chunks)(x2, ei, dinv, recip)

    x_pad = jnp.pad(x, ((0, n_pad - n), (0, 0)))
    out = _tc_linear(n_pad, d_in, d_out, 1024)(
        x_pad, u_all, deg[:, None], W, b[None, :]
    )
    return out[:n]
